# baseline (speedup 1.0000x reference)
import jax
import jax.numpy as jnp
from jax import lax
from jax.experimental import pallas as pl
from jax.experimental.pallas import tpu as pltpu

D = 128
B_TOK = 1024 * 512
BM = 2048
NBLK = B_TOK // BM
VPAD = 256


def _tc_body(ids_ref, tab_ref, out_ref):
    ids = ids_ref[0, 0, :]
    oh = (ids[:, None] == lax.broadcasted_iota(jnp.int32, (BM, VPAD), 1))
    oh = oh.astype(jnp.bfloat16)
    out_ref[...] = jnp.dot(oh, tab_ref[...],
                           preferred_element_type=jnp.float32)


def kernel(input_ids, word_embeddings, text_embeddings):
    del text_embeddings
    ids = input_ids.reshape(NBLK, 1, BM)
    tab = jnp.pad(word_embeddings, ((0, VPAD - 178), (0, 0)))
    tab = tab.astype(jnp.bfloat16)
    out = pl.pallas_call(
        _tc_body,
        grid=(NBLK,),
        in_specs=[
            pl.BlockSpec((1, 1, BM), lambda i: (i, 0, 0)),
            pl.BlockSpec((VPAD, D), lambda i: (0, 0)),
        ],
        out_specs=pl.BlockSpec((BM, D), lambda i: (i, 0)),
        out_shape=jax.ShapeDtypeStruct((B_TOK, D), jnp.float32),
    )(ids, tab)
    return out.reshape(1024, 512, D)


# speedup vs baseline: 1.7006x; 1.7006x over previous
import functools

import jax
import jax.numpy as jnp
from jax import lax
from jax.experimental import pallas as pl
from jax.experimental.pallas import tpu as pltpu
from jax.experimental.pallas import tpu_sc as plsc

D = 128
VOCAB = 178
B_TOK = 1024 * 512
NC, NS = 2, 16
NW = NC * NS
K = 128
SB = 2 * K
NSB = B_TOK // (NW * SB)
NCH = 2 * NSB


def _body(idx_hbm, table_hbm, out_hbm, idx_v, rows_v, tab_v, tab_sh,
          gsem, wsem):
    sid = lax.axis_index("s")
    wid = sid * NC + lax.axis_index("c")

    @pl.when(sid == 0)
    def _():
        pltpu.sync_copy(table_hbm, tab_v)
        pltpu.sync_copy(tab_v, tab_sh)

    pltpu.sync_copy(idx_hbm.at[wid], idx_v)
    plsc.subcore_barrier()
    base = wid * (NSB * SB)

    def gather_copy(c, s, h):
        return pltpu.make_async_copy(tab_sh.at[idx_v.at[c]],
                                     rows_v.at[s].at[pl.ds(h * K, K)],
                                     gsem)

    def gathers_start(sc, s):
        gather_copy(2 * sc, s, 0).start()
        gather_copy(2 * sc + 1, s, 1).start()

    def gathers_wait(sc, s):
        gather_copy(2 * sc, s, 0).wait()
        gather_copy(2 * sc + 1, s, 1).wait()

    def write_copy(sc, s):
        return pltpu.make_async_copy(rows_v.at[s],
                                     out_hbm.at[pl.ds(base + sc * SB, SB)],
                                     wsem)

    gathers_start(0, 0)
    gathers_start(1, 1)
    gathers_wait(0, 0)
    write_copy(0, 0).start()

    def outer(i, carry):
        sc1 = 2 * i + 1
        sc2 = 2 * i + 2
        gathers_wait(sc1, 1)
        write_copy(sc1, 1).start()
        write_copy(sc1 - 1, 0).wait()
        gathers_start(sc2, 0)
        gathers_wait(sc2, 0)
        write_copy(sc2, 0).start()
        write_copy(sc1, 1).wait()
        gathers_start(sc2 + 1, 1)
        return carry

    lax.fori_loop(0, NSB // 2 - 1, outer, 0)

    gathers_wait(NSB - 1, 1)
    write_copy(NSB - 1, 1).start()
    write_copy(NSB - 2, 0).wait()
    write_copy(NSB - 1, 1).wait()


def kernel(input_ids, word_embeddings, text_embeddings):
    del text_embeddings
    idx = input_ids.reshape(NW, NCH, K)
    run = functools.partial(
        pl.kernel,
        mesh=plsc.VectorSubcoreMesh(core_axis_name="c", subcore_axis_name="s"),
        out_type=jax.ShapeDtypeStruct((B_TOK, D), jnp.float32),
        scratch_types=[
            pltpu.VMEM((NCH, K), jnp.int32),
            pltpu.VMEM((2, SB, D), jnp.float32),
            pltpu.VMEM((VOCAB, D), jnp.float32),
            pltpu.VMEM_SHARED((VOCAB, D), jnp.float32),
            pltpu.SemaphoreType.DMA,
            pltpu.SemaphoreType.DMA,
        ],
    )(_body)
    out = run(idx, word_embeddings)
    return out.reshape(1024, 512, D)


# speedup vs baseline: 1.7716x; 1.0418x over previous
import functools

import jax
import jax.numpy as jnp
from jax import lax
from jax.experimental import pallas as pl
from jax.experimental.pallas import tpu as pltpu
from jax.experimental.pallas import tpu_sc as plsc

D = 128
B_TOK = 1024 * 512
NC, NS = 2, 16
NW = NC * NS
K = 128
CHUNKS = B_TOK // (NW * K)
NBUF = 4


def _body(idx_hbm, table_hbm, out_hbm, idx_v, rows_v, tab_v, tab_sh,
          gsem, wsem):
    sid = lax.axis_index("s")
    wid = sid * NC + lax.axis_index("c")

    @pl.when(sid == 0)
    def _():
        pltpu.sync_copy(table_hbm, tab_v)
        pltpu.sync_copy(tab_v, tab_sh)

    pltpu.sync_copy(idx_hbm.at[wid], idx_v)
    plsc.subcore_barrier()
    base = wid * (CHUNKS * K)

    def gather_copy(j, b):
        return pltpu.make_async_copy(tab_sh.at[idx_v.at[j]],
                                     rows_v.at[b], gsem)

    def write_copy(j, b):
        return pltpu.make_async_copy(rows_v.at[b],
                                     out_hbm.at[pl.ds(base + j * K, K)],
                                     wsem)

    gather_copy(0, 0).start()
    gather_copy(1, 1).start()
    gather_copy(2, 2).start()
    gather_copy(0, 0).wait()
    write_copy(0, 0).start()
    gather_copy(3, 3).start()
    gather_copy(1, 1).wait()
    write_copy(1, 1).start()

    def outer(i, carry):
        for b in range(NBUF):
            j = i * NBUF + b
            write_copy(j - NBUF, b).wait()
            gather_copy(j, b).start()
            pb = (b + 2) % NBUF
            gather_copy(j - 2, pb).wait()
            write_copy(j - 2, pb).start()
        return carry

    lax.fori_loop(1, CHUNKS // NBUF, outer, 0)

    gather_copy(CHUNKS - 2, 2).wait()
    write_copy(CHUNKS - 2, 2).start()
    gather_copy(CHUNKS - 1, 3).wait()
    write_copy(CHUNKS - 1, 3).start()
    for b in range(NBUF):
        write_copy(CHUNKS - NBUF + b, b).wait()


def kernel(input_ids, word_embeddings, text_embeddings):
    del text_embeddings
    idx = input_ids.reshape(NW, CHUNKS, K)
    run = functools.partial(
        pl.kernel,
        mesh=plsc.VectorSubcoreMesh(core_axis_name="c", subcore_axis_name="s"),
        out_type=jax.ShapeDtypeStruct((B_TOK, D), jnp.float32),
        scratch_types=[
            pltpu.VMEM((CHUNKS, K), jnp.int32),
            pltpu.VMEM((NBUF, K, D), jnp.float32),
            pltpu.VMEM((178, D), jnp.float32),
            pltpu.VMEM_SHARED((178, D), jnp.float32),
            pltpu.SemaphoreType.DMA,
            pltpu.SemaphoreType.DMA,
        ],
    )(_body)
    out = run(idx, word_embeddings)
    return out.reshape(1024, 512, D)
